# initial kernel scaffold (unmeasured)
import jax
import jax.numpy as jnp
from jax import lax
from jax.experimental import pallas as pl
from jax.experimental.pallas import tpu as pltpu

N_DEV = 4
SQ = 512
D = 1024
HQ = 8
DH = 128
SKV = 2048
SCALE = 0.08838834764831843


def kernel(x, Wq, Wo, K_ext, V_ext):
    my_i = lax.axis_index("i")
    xs = x.reshape(SQ, D).astype(jnp.bfloat16)
    wq = Wq.astype(jnp.bfloat16)
    wo = Wo.astype(jnp.bfloat16)
    k_loc = lax.dynamic_slice_in_dim(K_ext, my_i * HQ, HQ, axis=2).astype(
        jnp.bfloat16
    )
    v_loc = lax.dynamic_slice_in_dim(V_ext, my_i * HQ, HQ, axis=2).astype(
        jnp.bfloat16
    )

    def body(
        x_ref, wq_ref, wo_ref, k_ref, v_ref, out_ref,
        xall, partial, rs_buf, attn,
        ag_send, ag_recv, rs_send, rs_recv,
    ):
        me = lax.axis_index("i")
        left = (me - 1) % N_DEV
        right = (me + 1) % N_DEV

        barrier = pltpu.get_barrier_semaphore()
        for nbr in (left, right):
            pl.semaphore_signal(
                barrier, inc=1,
                device_id=(nbr,), device_id_type=pl.DeviceIdType.MESH,
            )
        pl.semaphore_wait(barrier, 2)

        pl.store(
            xall, (pl.ds(me, 1), slice(None), slice(None)),
            x_ref[...][None],
        )
        for h in range(N_DEV - 1):
            chunk = (me - h) % N_DEV
            rdma = pltpu.make_async_remote_copy(
                src_ref=xall.at[pl.ds(chunk, 1)],
                dst_ref=xall.at[pl.ds(chunk, 1)],
                send_sem=ag_send.at[h],
                recv_sem=ag_recv.at[h],
                device_id=(right,),
                device_id_type=pl.DeviceIdType.MESH,
            )
            rdma.start()
            rdma.wait()

        for b in range(N_DEV):
            xb = xall[b]
            q32 = jnp.dot(xb, wq_ref[...], preferred_element_type=jnp.float32)
            qb = q32.astype(jnp.bfloat16)
            for h in range(HQ):
                q = qb[:, h * DH:(h + 1) * DH]
                k = k_ref[b, :, h, :]
                v = v_ref[b, :, h, :]
                s = lax.dot_general(
                    q, k, (((1,), (1,)), ((), ())),
                    preferred_element_type=jnp.float32,
                ) * SCALE
                m = jnp.max(s, axis=1, keepdims=True)
                p = jnp.exp(s - m)
                l = jnp.sum(p, axis=1, keepdims=True)
                o = jnp.dot(
                    p.astype(jnp.bfloat16), v,
                    preferred_element_type=jnp.float32,
                ) / l
                attn[:, h * DH:(h + 1) * DH] = o.astype(jnp.bfloat16)
            pb = jnp.dot(attn[...], wo_ref[...],
                         preferred_element_type=jnp.float32)
            partial[b] = pb

        for s_ in range(N_DEV - 1):
            if s_ == 0:
                src = partial.at[pl.ds((me - 1) % N_DEV, 1)]
            else:
                src = rs_buf.at[pl.ds(s_ - 1, 1)]
            rdma = pltpu.make_async_remote_copy(
                src_ref=src,
                dst_ref=rs_buf.at[pl.ds(s_, 1)],
                send_sem=rs_send.at[s_],
                recv_sem=rs_recv.at[s_],
                device_id=(right,),
                device_id_type=pl.DeviceIdType.MESH,
            )
            rdma.start()
            rdma.wait()
            c = (me - 2 - s_) % N_DEV
            acc = pl.load(rs_buf, (pl.ds(s_, 1), slice(None), slice(None)))
            mine = pl.load(partial, (pl.ds(c, 1), slice(None), slice(None)))
            pl.store(
                rs_buf, (pl.ds(s_, 1), slice(None), slice(None)), acc + mine
            )

        out_ref[0] = rs_buf[N_DEV - 2]

    return pl.pallas_call(
        body,
        out_shape=jax.ShapeDtypeStruct((1, SQ, D), jnp.float32),
        in_specs=[pl.BlockSpec(memory_space=pltpu.VMEM)] * 5,
        out_specs=pl.BlockSpec(memory_space=pltpu.VMEM),
        scratch_shapes=[
            pltpu.VMEM((N_DEV, SQ, D), jnp.bfloat16),
            pltpu.VMEM((N_DEV, SQ, D), jnp.float32),
            pltpu.VMEM((N_DEV - 1, SQ, D), jnp.float32),
            pltpu.VMEM((SQ, D), jnp.bfloat16),
            pltpu.SemaphoreType.DMA((N_DEV - 1,)),
            pltpu.SemaphoreType.DMA((N_DEV - 1,)),
            pltpu.SemaphoreType.DMA((N_DEV - 1,)),
            pltpu.SemaphoreType.DMA((N_DEV - 1,)),
        ],
        compiler_params=pltpu.CompilerParams(collective_id=0),
    )(xs, wq, wo, k_loc, v_loc)


# baseline (device time: 295007 ns/iter reference)
import jax
import jax.numpy as jnp
from jax import lax
from jax.experimental import pallas as pl
from jax.experimental.pallas import tpu as pltpu

N_DEV = 4
SQ = 512
D = 1024
HQ = 8
DH = 128
SKV = 2048
SCALE = 0.08838834764831843


def kernel(x, Wq, Wo, K_ext, V_ext):
    my_i = lax.axis_index("i")
    xs = x.reshape(SQ, D).astype(jnp.bfloat16)
    wq3 = Wq.reshape(D, HQ, DH).transpose(1, 0, 2).astype(jnp.bfloat16)
    wo3 = Wo.reshape(HQ, DH, D).astype(jnp.bfloat16)
    k_loc = lax.dynamic_slice_in_dim(K_ext, my_i * HQ, HQ, axis=2)
    v_loc = lax.dynamic_slice_in_dim(V_ext, my_i * HQ, HQ, axis=2)
    k_loc = k_loc.transpose(0, 2, 1, 3).astype(jnp.bfloat16)
    v_loc = v_loc.transpose(0, 2, 1, 3).astype(jnp.bfloat16)

    def body(
        x_ref, wq_ref, wo_ref, k_ref, v_ref, out_ref,
        xall, partial, rs_buf, kbuf, vbuf,
        ag_send, ag_recv, rs_send, rs_recv, sem_k, sem_v,
    ):
        me = lax.axis_index("i")
        left = (me - 1) % N_DEV
        right = (me + 1) % N_DEV

        barrier = pltpu.get_barrier_semaphore()
        for nbr in (left, right):
            pl.semaphore_signal(
                barrier, inc=1,
                device_id=(nbr,), device_id_type=pl.DeviceIdType.MESH,
            )
        pl.semaphore_wait(barrier, 2)

        xall[pl.ds(me, 1)] = x_ref[...][None]
        for h in range(N_DEV - 1):
            chunk = (me - h) % N_DEV
            rdma = pltpu.make_async_remote_copy(
                src_ref=xall.at[pl.ds(chunk, 1)],
                dst_ref=xall.at[pl.ds(chunk, 1)],
                send_sem=ag_send.at[h],
                recv_sem=ag_recv.at[h],
                device_id=(right,),
                device_id_type=pl.DeviceIdType.MESH,
            )
            rdma.start()
            rdma.wait()

        for b in range(N_DEV):
            ck = pltpu.make_async_copy(k_ref.at[b], kbuf, sem_k)
            cv = pltpu.make_async_copy(v_ref.at[b], vbuf, sem_v)
            ck.start()
            cv.start()
            xb = xall[b]
            ck.wait()
            cv.wait()

            def head_body(h, pb):
                wq_h = wq_ref[pl.ds(h, 1)][0]
                q = jnp.dot(xb, wq_h,
                            preferred_element_type=jnp.float32)
                q = q.astype(jnp.bfloat16)
                k = kbuf[pl.ds(h, 1)][0]
                v = vbuf[pl.ds(h, 1)][0]
                s = lax.dot_general(
                    q, k, (((1,), (1,)), ((), ())),
                    preferred_element_type=jnp.float32,
                ) * SCALE
                m = jnp.max(s, axis=1, keepdims=True)
                p = jnp.exp(s - m)
                l = jnp.sum(p, axis=1, keepdims=True)
                o = jnp.dot(
                    p.astype(jnp.bfloat16), v,
                    preferred_element_type=jnp.float32,
                ) / l
                wo_h = wo_ref[pl.ds(h, 1)][0]
                return pb + jnp.dot(o.astype(jnp.bfloat16), wo_h,
                                    preferred_element_type=jnp.float32)

            pb = lax.fori_loop(
                0, HQ, head_body, jnp.zeros((SQ, D), jnp.float32)
            )
            partial[b] = pb.astype(jnp.bfloat16)

        for s_ in range(N_DEV - 1):
            if s_ == 0:
                src = partial.at[pl.ds((me - 1) % N_DEV, 1)]
            else:
                src = rs_buf.at[pl.ds(s_ - 1, 1)]
            rdma = pltpu.make_async_remote_copy(
                src_ref=src,
                dst_ref=rs_buf.at[pl.ds(s_, 1)],
                send_sem=rs_send.at[s_],
                recv_sem=rs_recv.at[s_],
                device_id=(right,),
                device_id_type=pl.DeviceIdType.MESH,
            )
            rdma.start()
            rdma.wait()
            c = (me - 2 - s_) % N_DEV
            acc = (rs_buf[s_].astype(jnp.float32)
                   + partial[pl.ds(c, 1)][0].astype(jnp.float32))
            if s_ < N_DEV - 2:
                rs_buf[s_] = acc.astype(jnp.bfloat16)
            else:
                out_ref[0] = acc

    return pl.pallas_call(
        body,
        out_shape=jax.ShapeDtypeStruct((1, SQ, D), jnp.float32),
        in_specs=[
            pl.BlockSpec(memory_space=pltpu.VMEM),
            pl.BlockSpec(memory_space=pltpu.VMEM),
            pl.BlockSpec(memory_space=pltpu.VMEM),
            pl.BlockSpec(memory_space=pl.ANY),
            pl.BlockSpec(memory_space=pl.ANY),
        ],
        out_specs=pl.BlockSpec(memory_space=pltpu.VMEM),
        scratch_shapes=[
            pltpu.VMEM((N_DEV, SQ, D), jnp.bfloat16),
            pltpu.VMEM((N_DEV, SQ, D), jnp.bfloat16),
            pltpu.VMEM((N_DEV - 1, SQ, D), jnp.bfloat16),
            pltpu.VMEM((HQ, SKV, DH), jnp.bfloat16),
            pltpu.VMEM((HQ, SKV, DH), jnp.bfloat16),
            pltpu.SemaphoreType.DMA((N_DEV - 1,)),
            pltpu.SemaphoreType.DMA((N_DEV - 1,)),
            pltpu.SemaphoreType.DMA((N_DEV - 1,)),
            pltpu.SemaphoreType.DMA((N_DEV - 1,)),
            pltpu.SemaphoreType.DMA,
            pltpu.SemaphoreType.DMA,
        ],
        compiler_params=pltpu.CompilerParams(
            collective_id=0,
            vmem_limit_bytes=40 * 1024 * 1024,
        ),
    )(xs, wq3, wo3, k_loc, v_loc)


# device time: 186695 ns/iter; 1.5802x vs baseline; 1.5802x over previous
import jax
import jax.numpy as jnp
from jax import lax
from jax.experimental import pallas as pl
from jax.experimental.pallas import tpu as pltpu

N_DEV = 4
SQ = 512
D = 1024
HQ = 8
DH = 128
SKV = 2048
SCALE = 0.08838834764831843


def kernel(x, Wq, Wo, K_ext, V_ext):
    my_i = lax.axis_index("i")
    xs = x.reshape(SQ, D).astype(jnp.bfloat16)
    wq3 = (Wq * SCALE).reshape(D, HQ, DH).transpose(1, 0, 2).astype(
        jnp.bfloat16
    )
    wo3 = Wo.reshape(HQ, DH, D).astype(jnp.bfloat16)
    k_loc = lax.dynamic_slice_in_dim(K_ext, my_i * HQ, HQ, axis=2)
    v_loc = lax.dynamic_slice_in_dim(V_ext, my_i * HQ, HQ, axis=2)
    k_loc = k_loc.transpose(0, 2, 1, 3).astype(jnp.bfloat16)
    v_loc = v_loc.transpose(0, 2, 1, 3).astype(jnp.bfloat16)

    def body(
        x_ref, wq_ref, wo_ref, k_ref, v_ref, out_ref,
        xall, partial, rs_buf, kbuf, vbuf,
        ag_send, ag_recv, rs_send, rs_recv, sem_k, sem_v,
    ):
        me = lax.axis_index("i")
        left = (me - 1) % N_DEV
        right = (me + 1) % N_DEV

        barrier = pltpu.get_barrier_semaphore()
        for nbr in (left, right):
            pl.semaphore_signal(
                barrier, inc=1,
                device_id=(nbr,), device_id_type=pl.DeviceIdType.MESH,
            )
        pl.semaphore_wait(barrier, 2)

        def ag_rdma(h):
            chunk = (me - h) % N_DEV
            return pltpu.make_async_remote_copy(
                src_ref=xall.at[pl.ds(chunk, 1)],
                dst_ref=xall.at[pl.ds(chunk, 1)],
                send_sem=ag_send.at[h],
                recv_sem=ag_recv.at[h],
                device_id=(right,),
                device_id_type=pl.DeviceIdType.MESH,
            )

        def rs_rdma(s_):
            if s_ == 0:
                src = partial.at[pl.ds((me - 1) % N_DEV, 1)]
            else:
                src = rs_buf.at[pl.ds(s_ - 1, 1)]
            return pltpu.make_async_remote_copy(
                src_ref=src,
                dst_ref=rs_buf.at[pl.ds(s_, 1)],
                send_sem=rs_send.at[s_],
                recv_sem=rs_recv.at[s_],
                device_id=(right,),
                device_id_type=pl.DeviceIdType.MESH,
            )

        def kv_dma(j):
            bj = (me - j) % N_DEV
            slot = j % 2
            ck = pltpu.make_async_copy(
                k_ref.at[pl.ds(bj, 1)], kbuf.at[pl.ds(slot, 1)],
                sem_k.at[slot],
            )
            cv = pltpu.make_async_copy(
                v_ref.at[pl.ds(bj, 1)], vbuf.at[pl.ds(slot, 1)],
                sem_v.at[slot],
            )
            return ck, cv

        def compute_batch(j):
            bj = (me - j) % N_DEV
            kb = kbuf.at[j % 2]
            vb = vbuf.at[j % 2]
            xb = xall[pl.ds(bj, 1)][0]

            def head_body(h, pb):
                wq_h = wq_ref[pl.ds(h, 1)][0]
                q = jnp.dot(xb, wq_h,
                            preferred_element_type=jnp.float32)
                q = q.astype(jnp.bfloat16)
                k = kb[pl.ds(h, 1)][0]
                v = vb[pl.ds(h, 1)][0]
                s = lax.dot_general(
                    q, k, (((1,), (1,)), ((), ())),
                    preferred_element_type=jnp.float32,
                )
                p = jnp.exp(s)
                l = jnp.sum(p, axis=1, keepdims=True)
                o = jnp.dot(
                    p.astype(jnp.bfloat16), v,
                    preferred_element_type=jnp.float32,
                ) / l
                wo_h = wo_ref[pl.ds(h, 1)][0]
                return pb + jnp.dot(o.astype(jnp.bfloat16), wo_h,
                                    preferred_element_type=jnp.float32)

            pb = lax.fori_loop(
                0, HQ, head_body, jnp.zeros((SQ, D), jnp.float32)
            )
            partial[pl.ds(bj, 1)] = pb.astype(jnp.bfloat16)[None]

        xall[pl.ds(me, 1)] = x_ref[...][None]
        ag0 = ag_rdma(0)
        ag0.start()
        ck0, cv0 = kv_dma(0)
        ck0.start()
        cv0.start()
        ck1, cv1 = kv_dma(1)
        ck1.start()
        cv1.start()
        ck0.wait()
        cv0.wait()
        compute_batch(0)

        ag0.wait()
        ag1 = ag_rdma(1)
        ag1.start()
        ck2, cv2 = kv_dma(2)
        ck2.start()
        cv2.start()
        ck1.wait()
        cv1.wait()
        compute_batch(1)
        rs0 = rs_rdma(0)
        rs0.start()

        ag1.wait()
        ag2 = ag_rdma(2)
        ag2.start()
        ck3, cv3 = kv_dma(3)
        ck3.start()
        cv3.start()
        ck2.wait()
        cv2.wait()
        compute_batch(2)
        rs0.wait()
        rs_buf[0] = (
            rs_buf[0].astype(jnp.float32)
            + partial[pl.ds((me - 2) % N_DEV, 1)][0].astype(jnp.float32)
        ).astype(jnp.bfloat16)
        rs1 = rs_rdma(1)
        rs1.start()

        ag2.wait()
        ck3.wait()
        cv3.wait()
        compute_batch(3)
        rs1.wait()
        rs_buf[1] = (
            rs_buf[1].astype(jnp.float32)
            + partial[pl.ds((me - 3) % N_DEV, 1)][0].astype(jnp.float32)
        ).astype(jnp.bfloat16)
        rs2 = rs_rdma(2)
        rs2.start()
        rs2.wait()
        out_ref[0] = (
            rs_buf[2].astype(jnp.float32)
            + partial[pl.ds(me, 1)][0].astype(jnp.float32)
        )

    return pl.pallas_call(
        body,
        out_shape=jax.ShapeDtypeStruct((1, SQ, D), jnp.float32),
        in_specs=[
            pl.BlockSpec(memory_space=pltpu.VMEM),
            pl.BlockSpec(memory_space=pltpu.VMEM),
            pl.BlockSpec(memory_space=pltpu.VMEM),
            pl.BlockSpec(memory_space=pl.ANY),
            pl.BlockSpec(memory_space=pl.ANY),
        ],
        out_specs=pl.BlockSpec(memory_space=pltpu.VMEM),
        scratch_shapes=[
            pltpu.VMEM((N_DEV, SQ, D), jnp.bfloat16),
            pltpu.VMEM((N_DEV, SQ, D), jnp.bfloat16),
            pltpu.VMEM((N_DEV - 1, SQ, D), jnp.bfloat16),
            pltpu.VMEM((2, HQ, SKV, DH), jnp.bfloat16),
            pltpu.VMEM((2, HQ, SKV, DH), jnp.bfloat16),
            pltpu.SemaphoreType.DMA((N_DEV - 1,)),
            pltpu.SemaphoreType.DMA((N_DEV - 1,)),
            pltpu.SemaphoreType.DMA((N_DEV - 1,)),
            pltpu.SemaphoreType.DMA((N_DEV - 1,)),
            pltpu.SemaphoreType.DMA((2,)),
            pltpu.SemaphoreType.DMA((2,)),
        ],
        compiler_params=pltpu.CompilerParams(
            collective_id=0,
            vmem_limit_bytes=36 * 1024 * 1024,
        ),
    )(xs, wq3, wo3, k_loc, v_loc)


# device time: 167292 ns/iter; 1.7634x vs baseline; 1.1160x over previous
import jax
import jax.numpy as jnp
from jax import lax
from jax.experimental import pallas as pl
from jax.experimental.pallas import tpu as pltpu

N_DEV = 4
SQ = 512
D = 1024
HQ = 8
DH = 128
SKV = 2048
SCALE = 0.08838834764831843


def kernel(x, Wq, Wo, K_ext, V_ext):
    xs = x.reshape(SQ, D).astype(jnp.bfloat16)
    wq3 = (Wq * SCALE).reshape(D, HQ, DH).transpose(1, 0, 2).astype(
        jnp.bfloat16
    )
    wo3 = Wo.reshape(HQ, DH, D).astype(jnp.bfloat16)

    def body(
        x_ref, wq_ref, wo_ref, k_ref, v_ref, out_ref,
        xall, partial, rs_buf, kbuf, vbuf,
        ag_send, ag_recv, rs_send, rs_recv, sem_k, sem_v,
    ):
        me = lax.axis_index("i")
        left = (me - 1) % N_DEV
        right = (me + 1) % N_DEV
        h0 = me * HQ

        barrier = pltpu.get_barrier_semaphore()
        for nbr in (left, right):
            pl.semaphore_signal(
                barrier, inc=1,
                device_id=(nbr,), device_id_type=pl.DeviceIdType.MESH,
            )
        pl.semaphore_wait(barrier, 2)

        def ag_rdma(hop):
            chunk = (me - hop) % N_DEV
            return pltpu.make_async_remote_copy(
                src_ref=xall.at[pl.ds(chunk, 1)],
                dst_ref=xall.at[pl.ds(chunk, 1)],
                send_sem=ag_send.at[hop],
                recv_sem=ag_recv.at[hop],
                device_id=(right,),
                device_id_type=pl.DeviceIdType.MESH,
            )

        def rs_rdma(s_):
            if s_ == 0:
                src = partial.at[pl.ds((me - 1) % N_DEV, 1)]
            else:
                src = rs_buf.at[pl.ds(s_ - 1, 1)]
            return pltpu.make_async_remote_copy(
                src_ref=src,
                dst_ref=rs_buf.at[pl.ds(s_, 1)],
                send_sem=rs_send.at[s_],
                recv_sem=rs_recv.at[s_],
                device_id=(right,),
                device_id_type=pl.DeviceIdType.MESH,
            )

        def kv_dma(bj, h, slot):
            ck = pltpu.make_async_copy(
                k_ref.at[pl.ds(bj, 1), :, pl.ds(h0 + h, 1), :],
                kbuf.at[pl.ds(slot, 1)],
                sem_k.at[slot],
            )
            cv = pltpu.make_async_copy(
                v_ref.at[pl.ds(bj, 1), :, pl.ds(h0 + h, 1), :],
                vbuf.at[pl.ds(slot, 1)],
                sem_v.at[slot],
            )
            return ck, cv

        def compute_batch(j):
            bj = (me - j) % N_DEV
            xb = xall[pl.ds(bj, 1)][0]

            def head_body(h, pb):
                slot = h % 2
                nslot = (h + 1) % 2

                @pl.when(h < HQ - 1)
                def _():
                    nk, nv = kv_dma(bj, h + 1, nslot)
                    nk.start()
                    nv.start()

                ck, cv = kv_dma(bj, h, slot)
                ck.wait()
                cv.wait()
                k = kbuf[pl.ds(slot, 1)][0, :, 0, :].astype(jnp.bfloat16)
                v = vbuf[pl.ds(slot, 1)][0, :, 0, :].astype(jnp.bfloat16)

                wq_h = wq_ref[pl.ds(h, 1)][0]
                q = jnp.dot(xb, wq_h,
                            preferred_element_type=jnp.float32)
                q = q.astype(jnp.bfloat16)
                s = lax.dot_general(
                    q, k, (((1,), (1,)), ((), ())),
                    preferred_element_type=jnp.float32,
                )
                p = jnp.exp(s)
                l = jnp.sum(p, axis=1, keepdims=True)
                o = jnp.dot(
                    p.astype(jnp.bfloat16), v,
                    preferred_element_type=jnp.float32,
                ) / l
                wo_h = wo_ref[pl.ds(h, 1)][0]
                return pb + jnp.dot(o.astype(jnp.bfloat16), wo_h,
                                    preferred_element_type=jnp.float32)

            ck0, cv0 = kv_dma(bj, 0, 0)
            ck0.start()
            cv0.start()
            pb = lax.fori_loop(
                0, HQ, head_body, jnp.zeros((SQ, D), jnp.float32)
            )
            partial[pl.ds(bj, 1)] = pb.astype(jnp.bfloat16)[None]

        xall[pl.ds(me, 1)] = x_ref[...][None]
        ag0 = ag_rdma(0)
        ag0.start()
        compute_batch(0)

        ag0.wait()
        ag1 = ag_rdma(1)
        ag1.start()
        compute_batch(1)
        rs0 = rs_rdma(0)
        rs0.start()

        ag1.wait()
        ag2 = ag_rdma(2)
        ag2.start()
        compute_batch(2)
        rs0.wait()
        rs_buf[0] = (
            rs_buf[0].astype(jnp.float32)
            + partial[pl.ds((me - 2) % N_DEV, 1)][0].astype(jnp.float32)
        ).astype(jnp.bfloat16)
        rs1 = rs_rdma(1)
        rs1.start()

        ag2.wait()
        compute_batch(3)
        rs1.wait()
        rs_buf[1] = (
            rs_buf[1].astype(jnp.float32)
            + partial[pl.ds((me - 3) % N_DEV, 1)][0].astype(jnp.float32)
        ).astype(jnp.bfloat16)
        rs2 = rs_rdma(2)
        rs2.start()
        rs2.wait()
        out_ref[0] = (
            rs_buf[2].astype(jnp.float32)
            + partial[pl.ds(me, 1)][0].astype(jnp.float32)
        )

    return pl.pallas_call(
        body,
        out_shape=jax.ShapeDtypeStruct((1, SQ, D), jnp.float32),
        in_specs=[
            pl.BlockSpec(memory_space=pltpu.VMEM),
            pl.BlockSpec(memory_space=pltpu.VMEM),
            pl.BlockSpec(memory_space=pltpu.VMEM),
            pl.BlockSpec(memory_space=pl.ANY),
            pl.BlockSpec(memory_space=pl.ANY),
        ],
        out_specs=pl.BlockSpec(memory_space=pltpu.VMEM),
        scratch_shapes=[
            pltpu.VMEM((N_DEV, SQ, D), jnp.bfloat16),
            pltpu.VMEM((N_DEV, SQ, D), jnp.bfloat16),
            pltpu.VMEM((N_DEV - 1, SQ, D), jnp.bfloat16),
            pltpu.VMEM((2, SKV, 1, DH), jnp.float32),
            pltpu.VMEM((2, SKV, 1, DH), jnp.float32),
            pltpu.SemaphoreType.DMA((N_DEV - 1,)),
            pltpu.SemaphoreType.DMA((N_DEV - 1,)),
            pltpu.SemaphoreType.DMA((N_DEV - 1,)),
            pltpu.SemaphoreType.DMA((N_DEV - 1,)),
            pltpu.SemaphoreType.DMA((2,)),
            pltpu.SemaphoreType.DMA((2,)),
        ],
        compiler_params=pltpu.CompilerParams(
            collective_id=0,
            vmem_limit_bytes=36 * 1024 * 1024,
        ),
    )(xs, wq3, wo3, K_ext, V_ext)


# device time: 164193 ns/iter; 1.7967x vs baseline; 1.0189x over previous
import jax
import jax.numpy as jnp
from jax import lax
from jax.experimental import pallas as pl
from jax.experimental.pallas import tpu as pltpu

N_DEV = 4
SQ = 512
D = 1024
HQ = 8
DH = 128
SKV = 2048
SCALE = 0.08838834764831843


def kernel(x, Wq, Wo, K_ext, V_ext):
    xs = x.reshape(SQ, D).astype(jnp.bfloat16)
    wq3 = (Wq * SCALE).reshape(D, HQ, DH).transpose(1, 0, 2).astype(
        jnp.bfloat16
    )
    wo3 = Wo.reshape(HQ, DH, D).astype(jnp.bfloat16)

    def body(
        x_ref, wq_ref, wo_ref, k_ref, v_ref, out_ref,
        xall, partial, rs_buf, kbuf, vbuf,
        ag_send, ag_recv, rs_send, rs_recv, sem_k, sem_v,
    ):
        me = lax.axis_index("i")
        left = (me - 1) % N_DEV
        right = (me + 1) % N_DEV
        h0 = me * HQ

        barrier = pltpu.get_barrier_semaphore()
        for nbr in (left, right):
            pl.semaphore_signal(
                barrier, inc=1,
                device_id=(nbr,), device_id_type=pl.DeviceIdType.MESH,
            )
        pl.semaphore_wait(barrier, 2)

        def ag_rdma(hop):
            chunk = (me - hop) % N_DEV
            return pltpu.make_async_remote_copy(
                src_ref=xall.at[pl.ds(chunk, 1)],
                dst_ref=xall.at[pl.ds(chunk, 1)],
                send_sem=ag_send.at[hop],
                recv_sem=ag_recv.at[hop],
                device_id=(right,),
                device_id_type=pl.DeviceIdType.MESH,
            )

        def rs_rdma(s_):
            if s_ == 0:
                src = partial.at[pl.ds((me - 1) % N_DEV, 1)]
            else:
                src = rs_buf.at[pl.ds(s_ - 1, 1)]
            return pltpu.make_async_remote_copy(
                src_ref=src,
                dst_ref=rs_buf.at[pl.ds(s_, 1)],
                send_sem=rs_send.at[s_],
                recv_sem=rs_recv.at[s_],
                device_id=(right,),
                device_id_type=pl.DeviceIdType.MESH,
            )

        def kv_dma(bj, h, slot):
            ck = pltpu.make_async_copy(
                k_ref.at[pl.ds(bj, 1), :, pl.ds(h0 + h, 1), :],
                kbuf.at[pl.ds(slot, 1)],
                sem_k.at[slot],
            )
            cv = pltpu.make_async_copy(
                v_ref.at[pl.ds(bj, 1), :, pl.ds(h0 + h, 1), :],
                vbuf.at[pl.ds(slot, 1)],
                sem_v.at[slot],
            )
            return ck, cv

        def compute_batch(j):
            bj = (me - j) % N_DEV
            xb = xall[pl.ds(bj, 1)][0]

            def one_head(h, slot):
                k = kbuf[pl.ds(slot, 1)][0, :, 0, :].astype(jnp.bfloat16)
                v = vbuf[pl.ds(slot, 1)][0, :, 0, :].astype(jnp.bfloat16)
                wq_h = wq_ref[pl.ds(h, 1)][0]
                q = jnp.dot(xb, wq_h,
                            preferred_element_type=jnp.float32)
                q = q.astype(jnp.bfloat16)
                s = lax.dot_general(
                    q, k, (((1,), (1,)), ((), ())),
                    preferred_element_type=jnp.float32,
                )
                p = jnp.exp(s)
                l = jnp.sum(p, axis=1, keepdims=True)
                o = jnp.dot(
                    p.astype(jnp.bfloat16), v,
                    preferred_element_type=jnp.float32,
                ) / l
                wo_h = wo_ref[pl.ds(h, 1)][0]
                return jnp.dot(o.astype(jnp.bfloat16), wo_h,
                               preferred_element_type=jnp.float32)

            def pair_body(hp, pb):
                base = 2 * (hp % 2)

                @pl.when(hp < HQ // 2 - 1)
                def _():
                    nbase = 2 * ((hp + 1) % 2)
                    for par in range(2):
                        nk, nv = kv_dma(bj, 2 * hp + 2 + par, nbase + par)
                        nk.start()
                        nv.start()

                for par in range(2):
                    ck, cv = kv_dma(bj, 2 * hp + par, base + par)
                    ck.wait()
                    cv.wait()
                pb = pb + one_head(2 * hp, base)
                pb = pb + one_head(2 * hp + 1, base + 1)
                return pb

            for par in range(2):
                ck0, cv0 = kv_dma(bj, par, par)
                ck0.start()
                cv0.start()
            pb = lax.fori_loop(
                0, HQ // 2, pair_body, jnp.zeros((SQ, D), jnp.float32)
            )
            partial[pl.ds(bj, 1)] = pb.astype(jnp.bfloat16)[None]

        xall[pl.ds(me, 1)] = x_ref[...][None]
        ag0 = ag_rdma(0)
        ag0.start()
        compute_batch(0)

        ag0.wait()
        ag1 = ag_rdma(1)
        ag1.start()
        compute_batch(1)
        rs0 = rs_rdma(0)
        rs0.start()

        ag1.wait()
        ag2 = ag_rdma(2)
        ag2.start()
        compute_batch(2)
        rs0.wait()
        rs_buf[0] = (
            rs_buf[0].astype(jnp.float32)
            + partial[pl.ds((me - 2) % N_DEV, 1)][0].astype(jnp.float32)
        ).astype(jnp.bfloat16)
        rs1 = rs_rdma(1)
        rs1.start()

        ag2.wait()
        compute_batch(3)
        rs1.wait()
        rs_buf[1] = (
            rs_buf[1].astype(jnp.float32)
            + partial[pl.ds((me - 3) % N_DEV, 1)][0].astype(jnp.float32)
        ).astype(jnp.bfloat16)
        rs2 = rs_rdma(2)
        rs2.start()
        rs2.wait()
        out_ref[0] = (
            rs_buf[2].astype(jnp.float32)
            + partial[pl.ds(me, 1)][0].astype(jnp.float32)
        )

    return pl.pallas_call(
        body,
        out_shape=jax.ShapeDtypeStruct((1, SQ, D), jnp.float32),
        in_specs=[
            pl.BlockSpec(memory_space=pltpu.VMEM),
            pl.BlockSpec(memory_space=pltpu.VMEM),
            pl.BlockSpec(memory_space=pltpu.VMEM),
            pl.BlockSpec(memory_space=pl.ANY),
            pl.BlockSpec(memory_space=pl.ANY),
        ],
        out_specs=pl.BlockSpec(memory_space=pltpu.VMEM),
        scratch_shapes=[
            pltpu.VMEM((N_DEV, SQ, D), jnp.bfloat16),
            pltpu.VMEM((N_DEV, SQ, D), jnp.bfloat16),
            pltpu.VMEM((N_DEV - 1, SQ, D), jnp.bfloat16),
            pltpu.VMEM((4, SKV, 1, DH), jnp.float32),
            pltpu.VMEM((4, SKV, 1, DH), jnp.float32),
            pltpu.SemaphoreType.DMA((N_DEV - 1,)),
            pltpu.SemaphoreType.DMA((N_DEV - 1,)),
            pltpu.SemaphoreType.DMA((N_DEV - 1,)),
            pltpu.SemaphoreType.DMA((N_DEV - 1,)),
            pltpu.SemaphoreType.DMA((4,)),
            pltpu.SemaphoreType.DMA((4,)),
        ],
        compiler_params=pltpu.CompilerParams(
            collective_id=0,
            vmem_limit_bytes=36 * 1024 * 1024,
        ),
    )(xs, wq3, wo3, K_ext, V_ext)
